# Initial kernel scaffold; baseline (speedup 1.0000x reference)
import functools

import jax
import jax.numpy as jnp
from jax import lax
from jax.experimental import pallas as pl
from jax.experimental.pallas import tpu as pltpu
from jax.experimental.pallas import tpu_sc as plsc

N = 10000
E = 320000
D = 128
NC = 2
NS = 16
L = 16
NW = NC * NS
EPT = E // NW
CH = 80
NCHUNK = EPT // CH
RPT = N // NS


def _sc_round(h, src, dst, ew):
    mesh = plsc.VectorSubcoreMesh(core_axis_name="c", subcore_axis_name="s")

    @functools.partial(
        pl.kernel,
        out_type=jax.ShapeDtypeStruct((NC, N, D), jnp.float32),
        mesh=mesh,
        scratch_types=[
            pltpu.VMEM_SHARED((N, D), jnp.float32),
            pltpu.VMEM((CH, D), jnp.float32),
            pltpu.VMEM((CH,), jnp.int32),
            pltpu.VMEM((CH,), jnp.int32),
            pltpu.VMEM((CH,), jnp.float32),
        ],
    )
    def k(h_hbm, src_hbm, dst_hbm, ew_hbm, out_hbm, acc, rows, sidx, didx, wbuf):
        c = lax.axis_index("c")
        s = lax.axis_index("s")
        wid = c * NS + s

        @pl.loop(0, CH)
        def _zero_fill(r):
            for g in range(D // L):
                rows[r, pl.ds(g * L, L)] = jnp.zeros((L,), jnp.float32)

        for i in range(RPT // CH):
            pltpu.sync_copy(rows, acc.at[pl.ds(s * RPT + i * CH, CH)])
        rem = RPT % CH
        if rem:
            pltpu.sync_copy(
                rows.at[pl.ds(0, rem)],
                acc.at[pl.ds(s * RPT + (RPT // CH) * CH, rem)],
            )
        plsc.subcore_barrier()

        base0 = wid * EPT

        @pl.loop(0, NCHUNK)
        def _chunk(ci):
            base = base0 + ci * CH
            pltpu.sync_copy(src_hbm.at[pl.ds(base, CH)], sidx)
            pltpu.sync_copy(dst_hbm.at[pl.ds(base, CH)], didx)
            pltpu.sync_copy(ew_hbm.at[pl.ds(base, CH)], wbuf)
            pltpu.sync_copy(h_hbm.at[sidx], rows)

            @pl.loop(0, CH)
            def _scale(r):
                w16 = plsc.load_gather(
                    wbuf, [jnp.broadcast_to(r, (L,)).astype(jnp.int32)]
                )
                for g in range(D // L):
                    rows[r, pl.ds(g * L, L)] = rows[r, pl.ds(g * L, L)] * w16

            pltpu.sync_copy(rows, acc.at[didx], add=True)

        plsc.subcore_barrier()

        for i in range(RPT // CH):
            pltpu.sync_copy(
                acc.at[pl.ds(s * RPT + i * CH, CH)],
                out_hbm.at[c, pl.ds(s * RPT + i * CH, CH)],
            )
        if rem:
            pltpu.sync_copy(
                acc.at[pl.ds(s * RPT + (RPT // CH) * CH, rem)],
                out_hbm.at[c, pl.ds(s * RPT + (RPT // CH) * CH, rem)],
            )

    return k(h, src, dst, ew)


_BLK = 2000


def _tc_combine(p):
    def body(p_ref, o_ref):
        o_ref[...] = p_ref[0] + p_ref[1]

    return pl.pallas_call(
        body,
        out_shape=jax.ShapeDtypeStruct((N, D), jnp.float32),
        grid=(N // _BLK,),
        in_specs=[pl.BlockSpec((NC, _BLK, D), lambda i: (0, i, 0))],
        out_specs=pl.BlockSpec((_BLK, D), lambda i: (i, 0)),
    )(p)


def _tc_final(x, h1, q, W, b2):
    def body(x_ref, h1_ref, q_ref, w_ref, b_ref, o_ref):
        sm = (x_ref[...] + h1_ref[...] + q_ref[0] + q_ref[1]) * (1.0 / 3.0)
        o_ref[...] = (
            lax.dot_general(
                sm,
                w_ref[...],
                (((1,), (1,)), ((), ())),
                precision=lax.Precision.HIGHEST,
            )
            + b_ref[...]
        )

    return pl.pallas_call(
        body,
        out_shape=jax.ShapeDtypeStruct((N, D), jnp.float32),
        grid=(N // _BLK,),
        in_specs=[
            pl.BlockSpec((_BLK, D), lambda i: (i, 0)),
            pl.BlockSpec((_BLK, D), lambda i: (i, 0)),
            pl.BlockSpec((NC, _BLK, D), lambda i: (0, i, 0)),
            pl.BlockSpec((D, D), lambda i: (0, 0)),
            pl.BlockSpec((1, D), lambda i: (0, 0)),
        ],
        out_specs=pl.BlockSpec((_BLK, D), lambda i: (i, 0)),
    )(x, h1, q, W, b2)


def kernel(x, edge_index, edge_weight, W, b):
    src = edge_index[0].astype(jnp.int32)
    dst = edge_index[1].astype(jnp.int32)
    ew = edge_weight.astype(jnp.float32)
    p = _sc_round(x, src, dst, ew)
    h1 = _tc_combine(p)
    q = _sc_round(h1, src, dst, ew)
    return _tc_final(x, h1, q, W, b.reshape(1, D))


# speedup vs baseline: 3.4180x; 3.4180x over previous
import dataclasses
import functools

import jax
import jax.numpy as jnp
from jax import lax
from jax.experimental import pallas as pl
from jax.experimental.pallas import tpu as pltpu
from jax.experimental.pallas import tpu_sc as plsc

N = 10000
E = 320000
D = 128
NC = 2
NS = 16
L = 16
NW = NC * NS
EPT = E // NW
CH = 80
NCHUNK = EPT // CH
NRC = N // CH


def _sc_round(h, src, dst, ew):
    mesh = plsc.VectorSubcoreMesh(core_axis_name="c", subcore_axis_name="s")
    cp = pltpu.CompilerParams()
    if "needs_layout_passes" in pltpu.CompilerParams.__dataclass_fields__:
        cp = dataclasses.replace(cp, needs_layout_passes=False)

    @functools.partial(
        pl.kernel,
        compiler_params=cp,
        out_type=jax.ShapeDtypeStruct((NC, N, D), jnp.float32),
        mesh=mesh,
        scratch_types=[
            pltpu.VMEM_SHARED((N, D), jnp.float32),
            pltpu.VMEM((CH, D), jnp.float32),
            pltpu.VMEM((CH,), jnp.int32),
            pltpu.VMEM((CH,), jnp.int32),
            pltpu.VMEM((CH,), jnp.float32),
        ],
    )
    def k(h_hbm, src_hbm, dst_hbm, ew_hbm, out_hbm, acc, rows, sidx, didx, wbuf):
        c = lax.axis_index("c")
        s = lax.axis_index("s")
        wid = c * NS + s

        @pl.loop(0, CH)
        def _zero_fill(r):
            for g in range(D // L):
                rows[r, pl.ds(g * L, L)] = jnp.zeros((L,), jnp.float32)

        for i in range((NRC + NS - 1) // NS):
            j = i * NS + s

            @pl.when(j < NRC)
            def _():
                pltpu.sync_copy(rows, acc.at[pl.ds(j * CH, CH)])

        plsc.subcore_barrier()

        base0 = wid * EPT

        @pl.loop(0, NCHUNK)
        def _chunk(ci):
            base = base0 + ci * CH
            pltpu.sync_copy(src_hbm.at[pl.ds(base, CH)], sidx)
            pltpu.sync_copy(dst_hbm.at[pl.ds(base, CH)], didx)
            pltpu.sync_copy(ew_hbm.at[pl.ds(base, CH)], wbuf)
            pltpu.sync_copy(h_hbm.at[sidx], rows)

            @pl.loop(0, CH)
            def _scale(r):
                w16 = plsc.load_gather(
                    wbuf, [jnp.broadcast_to(r, (L,)).astype(jnp.int32)]
                )
                for g in range(D // L):
                    rows[r, pl.ds(g * L, L)] = rows[r, pl.ds(g * L, L)] * w16

            pltpu.sync_copy(rows, acc.at[didx], add=True)

        plsc.subcore_barrier()

        for i in range((NRC + NS - 1) // NS):
            j = i * NS + s

            @pl.when(j < NRC)
            def _():
                pltpu.sync_copy(
                    acc.at[pl.ds(j * CH, CH)],
                    out_hbm.at[c, pl.ds(j * CH, CH)],
                )

    return k(h, src, dst, ew)


_BLK = 2000


def _tc_combine(p):
    def body(p_ref, o_ref):
        o_ref[...] = p_ref[0] + p_ref[1]

    return pl.pallas_call(
        body,
        out_shape=jax.ShapeDtypeStruct((N, D), jnp.float32),
        grid=(N // _BLK,),
        in_specs=[pl.BlockSpec((NC, _BLK, D), lambda i: (0, i, 0))],
        out_specs=pl.BlockSpec((_BLK, D), lambda i: (i, 0)),
    )(p)


def _tc_final(x, h1, q, W, b2):
    def body(x_ref, h1_ref, q_ref, w_ref, b_ref, o_ref):
        sm = (x_ref[...] + h1_ref[...] + q_ref[0] + q_ref[1]) * (1.0 / 3.0)
        o_ref[...] = (
            lax.dot_general(
                sm,
                w_ref[...],
                (((1,), (1,)), ((), ())),
                precision=lax.Precision.HIGHEST,
            )
            + b_ref[...]
        )

    return pl.pallas_call(
        body,
        out_shape=jax.ShapeDtypeStruct((N, D), jnp.float32),
        grid=(N // _BLK,),
        in_specs=[
            pl.BlockSpec((_BLK, D), lambda i: (i, 0)),
            pl.BlockSpec((_BLK, D), lambda i: (i, 0)),
            pl.BlockSpec((NC, _BLK, D), lambda i: (0, i, 0)),
            pl.BlockSpec((D, D), lambda i: (0, 0)),
            pl.BlockSpec((1, D), lambda i: (0, 0)),
        ],
        out_specs=pl.BlockSpec((_BLK, D), lambda i: (i, 0)),
    )(x, h1, q, W, b2)


def kernel(x, edge_index, edge_weight, W, b):
    src = edge_index[0].astype(jnp.int32)
    dst = edge_index[1].astype(jnp.int32)
    ew = edge_weight.astype(jnp.float32)
    p = _sc_round(x, src, dst, ew)
    h1 = _tc_combine(p)
    q = _sc_round(h1, src, dst, ew)
    return _tc_final(x, h1, q, W, b.reshape(1, D))


# speedup vs baseline: 8.3763x; 2.4507x over previous
import dataclasses
import functools

import jax
import jax.numpy as jnp
from jax import lax
from jax.experimental import pallas as pl
from jax.experimental.pallas import tpu as pltpu
from jax.experimental.pallas import tpu_sc as plsc

N = 10000
E = 320000
D = 128
NC = 2
NS = 16
L = 16
NW = NC * NS
EPT = E // NW
CH = 80
NCHUNK = EPT // CH
NRC = N // CH


def _sc_round(h, src, dst, ew):
    mesh = plsc.VectorSubcoreMesh(core_axis_name="c", subcore_axis_name="s")
    cp = pltpu.CompilerParams()
    if "needs_layout_passes" in pltpu.CompilerParams.__dataclass_fields__:
        cp = dataclasses.replace(cp, needs_layout_passes=False)

    @functools.partial(
        pl.kernel,
        compiler_params=cp,
        out_type=jax.ShapeDtypeStruct((NC, N, D), jnp.float32),
        mesh=mesh,
        scratch_types=[
            pltpu.VMEM_SHARED((N, D), jnp.float32),
            pltpu.VMEM((CH, D), jnp.float32),
            pltpu.VMEM((CH, D), jnp.float32),
            pltpu.VMEM((EPT,), jnp.int32),
            pltpu.VMEM((CH,), jnp.int32),
            pltpu.VMEM((CH,), jnp.int32),
            pltpu.VMEM((EPT,), jnp.float32),
            pltpu.SemaphoreType.DMA,
            pltpu.SemaphoreType.DMA,
            pltpu.SemaphoreType.DMA,
            pltpu.SemaphoreType.DMA,
        ],
    )
    def k(h_hbm, src_hbm, dst_hbm, ew_hbm, out_hbm,
          acc, rows0, rows1, sidx, dbuf0, dbuf1, wbuf,
          gsem0, gsem1, dsem0, dsem1):
        c = lax.axis_index("c")
        s = lax.axis_index("s")
        wid = c * NS + s

        pltpu.sync_copy(src_hbm.at[pl.ds(wid * EPT, EPT)], sidx)
        pltpu.sync_copy(ew_hbm.at[pl.ds(wid * EPT, EPT)], wbuf)

        @pl.loop(0, CH)
        def _zero_fill(r):
            for g in range(D // L):
                rows0[r, pl.ds(g * L, L)] = jnp.zeros((L,), jnp.float32)

        for i in range((NRC + NS - 1) // NS):
            j = i * NS + s

            @pl.when(j < NRC)
            def _():
                pltpu.sync_copy(rows0, acc.at[pl.ds(j * CH, CH)])

        plsc.subcore_barrier()

        def gather_start(ci, buf, sem):
            pltpu.async_copy(h_hbm.at[sidx.at[pl.ds(ci * CH, CH)]], buf, sem)

        def gather_wait(ci, buf, sem):
            pltpu.make_async_copy(
                h_hbm.at[sidx.at[pl.ds(ci * CH, CH)]], buf, sem
            ).wait()

        def dst_start(ci, dbuf, sem):
            pltpu.async_copy(dst_hbm.at[pl.ds(wid * EPT + ci * CH, CH)], dbuf, sem)

        def dst_wait(ci, dbuf, sem):
            pltpu.make_async_copy(
                dst_hbm.at[pl.ds(wid * EPT + ci * CH, CH)], dbuf, sem
            ).wait()

        def scale(ci, buf):
            @pl.loop(0, CH)
            def _scale(r):
                w16 = plsc.load_gather(
                    wbuf, [jnp.broadcast_to(ci * CH + r, (L,)).astype(jnp.int32)]
                )
                for g in range(D // L):
                    buf[r, pl.ds(g * L, L)] = buf[r, pl.ds(g * L, L)] * w16

        def scatter_add(buf, dbuf):
            pltpu.sync_copy(buf, acc.at[dbuf], add=True)

        gather_start(0, rows0, gsem0)
        dst_start(0, dbuf0, dsem0)

        @pl.loop(0, NCHUNK - 1, step=2)
        def _pipe(ci):
            gather_start(ci + 1, rows1, gsem1)
            dst_start(ci + 1, dbuf1, dsem1)
            gather_wait(ci, rows0, gsem0)
            scale(ci, rows0)
            dst_wait(ci, dbuf0, dsem0)
            scatter_add(rows0, dbuf0)
            gather_start(ci + 2, rows0, gsem0)
            dst_start(ci + 2, dbuf0, dsem0)
            gather_wait(ci + 1, rows1, gsem1)
            scale(ci + 1, rows1)
            dst_wait(ci + 1, dbuf1, dsem1)
            scatter_add(rows1, dbuf1)

        gather_wait(NCHUNK - 1, rows0, gsem0)
        scale(NCHUNK - 1, rows0)
        dst_wait(NCHUNK - 1, dbuf0, dsem0)
        scatter_add(rows0, dbuf0)

        plsc.subcore_barrier()

        for i in range((NRC + NS - 1) // NS):
            j = i * NS + s

            @pl.when(j < NRC)
            def _():
                pltpu.sync_copy(
                    acc.at[pl.ds(j * CH, CH)],
                    out_hbm.at[c, pl.ds(j * CH, CH)],
                )

    return k(h, src, dst, ew)


_BLK = 2000


def _tc_combine(p):
    def body(p_ref, o_ref):
        o_ref[...] = p_ref[0] + p_ref[1]

    return pl.pallas_call(
        body,
        out_shape=jax.ShapeDtypeStruct((N, D), jnp.float32),
        grid=(N // _BLK,),
        in_specs=[pl.BlockSpec((NC, _BLK, D), lambda i: (0, i, 0))],
        out_specs=pl.BlockSpec((_BLK, D), lambda i: (i, 0)),
    )(p)


def _tc_final(x, h1, q, W, b2):
    def body(x_ref, h1_ref, q_ref, w_ref, b_ref, o_ref):
        sm = (x_ref[...] + h1_ref[...] + q_ref[0] + q_ref[1]) * (1.0 / 3.0)
        o_ref[...] = (
            lax.dot_general(
                sm,
                w_ref[...],
                (((1,), (1,)), ((), ())),
                precision=lax.Precision.HIGHEST,
            )
            + b_ref[...]
        )

    return pl.pallas_call(
        body,
        out_shape=jax.ShapeDtypeStruct((N, D), jnp.float32),
        grid=(N // _BLK,),
        in_specs=[
            pl.BlockSpec((_BLK, D), lambda i: (i, 0)),
            pl.BlockSpec((_BLK, D), lambda i: (i, 0)),
            pl.BlockSpec((NC, _BLK, D), lambda i: (0, i, 0)),
            pl.BlockSpec((D, D), lambda i: (0, 0)),
            pl.BlockSpec((1, D), lambda i: (0, 0)),
        ],
        out_specs=pl.BlockSpec((_BLK, D), lambda i: (i, 0)),
    )(x, h1, q, W, b2)


def kernel(x, edge_index, edge_weight, W, b):
    src = edge_index[0].astype(jnp.int32)
    dst = edge_index[1].astype(jnp.int32)
    ew = edge_weight.astype(jnp.float32)
    p = _sc_round(x, src, dst, ew)
    h1 = _tc_combine(p)
    q = _sc_round(h1, src, dst, ew)
    return _tc_final(x, h1, q, W, b.reshape(1, D))


# speedup vs baseline: 10.0632x; 1.2014x over previous
import dataclasses
import functools

import jax
import jax.numpy as jnp
from jax import lax
from jax.experimental import pallas as pl
from jax.experimental.pallas import tpu as pltpu
from jax.experimental.pallas import tpu_sc as plsc

N = 10000
E = 320000
D = 128
NC = 2
NS = 16
L = 16
NW = NC * NS
EPT = E // NW
CH = 80
NCHUNK = EPT // CH
NRC = N // CH


def _sc_round(h, src, dst, ew):
    mesh = plsc.VectorSubcoreMesh(core_axis_name="c", subcore_axis_name="s")
    cp = pltpu.CompilerParams()
    if "needs_layout_passes" in pltpu.CompilerParams.__dataclass_fields__:
        cp = dataclasses.replace(cp, needs_layout_passes=False)

    @functools.partial(
        pl.kernel,
        compiler_params=cp,
        out_type=jax.ShapeDtypeStruct((NC, N, D), jnp.float32),
        mesh=mesh,
        scratch_types=[
            pltpu.VMEM_SHARED((N, D), jnp.float32),
            pltpu.VMEM((CH, D), jnp.float32),
            pltpu.VMEM((CH, D), jnp.float32),
            pltpu.VMEM((EPT,), jnp.int32),
            pltpu.VMEM((CH,), jnp.int32),
            pltpu.VMEM((CH,), jnp.int32),
            pltpu.VMEM((EPT,), jnp.float32),
            pltpu.SemaphoreType.DMA,
            pltpu.SemaphoreType.DMA,
            pltpu.SemaphoreType.DMA,
            pltpu.SemaphoreType.DMA,
        ],
    )
    def k(h_hbm, src_hbm, dst_hbm, ew_hbm, out_hbm,
          acc, rows0, rows1, sidx, dbuf0, dbuf1, wbuf,
          gsem0, gsem1, dsem0, dsem1):
        c = lax.axis_index("c")
        s = lax.axis_index("s")
        wid = c * NS + s

        pltpu.sync_copy(src_hbm.at[pl.ds(wid * EPT, EPT)], sidx)
        pltpu.sync_copy(ew_hbm.at[pl.ds(wid * EPT, EPT)], wbuf)

        @pl.loop(0, CH)
        def _zero_fill(r):
            for g in range(D // L):
                rows0[r, pl.ds(g * L, L)] = jnp.zeros((L,), jnp.float32)

        for i in range((NRC + NS - 1) // NS):
            j = i * NS + s

            @pl.when(j < NRC)
            def _():
                pltpu.sync_copy(rows0, acc.at[pl.ds(j * CH, CH)])

        plsc.subcore_barrier()

        def gather_start(ci, buf, sem):
            pltpu.async_copy(h_hbm.at[sidx.at[pl.ds(ci * CH, CH)]], buf, sem)

        def gather_wait(ci, buf, sem):
            pltpu.make_async_copy(
                h_hbm.at[sidx.at[pl.ds(ci * CH, CH)]], buf, sem
            ).wait()

        def dst_start(ci, dbuf, sem):
            pltpu.async_copy(dst_hbm.at[pl.ds(wid * EPT + ci * CH, CH)], dbuf, sem)

        def dst_wait(ci, dbuf, sem):
            pltpu.make_async_copy(
                dst_hbm.at[pl.ds(wid * EPT + ci * CH, CH)], dbuf, sem
            ).wait()

        def scale(ci, buf):
            @plsc.parallel_loop(0, CH, unroll=4)
            def _scale(r):
                w16 = plsc.load_gather(
                    wbuf, [jnp.broadcast_to(ci * CH + r, (L,)).astype(jnp.int32)]
                )
                for g in range(D // L):
                    buf[r, pl.ds(g * L, L)] = buf[r, pl.ds(g * L, L)] * w16

        def scatter_add(buf, dbuf):
            pltpu.sync_copy(buf, acc.at[dbuf], add=True)

        gather_start(0, rows0, gsem0)
        dst_start(0, dbuf0, dsem0)

        @pl.loop(0, NCHUNK - 1, step=2)
        def _pipe(ci):
            gather_start(ci + 1, rows1, gsem1)
            dst_start(ci + 1, dbuf1, dsem1)
            gather_wait(ci, rows0, gsem0)
            scale(ci, rows0)
            dst_wait(ci, dbuf0, dsem0)
            scatter_add(rows0, dbuf0)
            gather_start(ci + 2, rows0, gsem0)
            dst_start(ci + 2, dbuf0, dsem0)
            gather_wait(ci + 1, rows1, gsem1)
            scale(ci + 1, rows1)
            dst_wait(ci + 1, dbuf1, dsem1)
            scatter_add(rows1, dbuf1)

        gather_wait(NCHUNK - 1, rows0, gsem0)
        scale(NCHUNK - 1, rows0)
        dst_wait(NCHUNK - 1, dbuf0, dsem0)
        scatter_add(rows0, dbuf0)

        plsc.subcore_barrier()

        for i in range((NRC + NS - 1) // NS):
            j = i * NS + s

            @pl.when(j < NRC)
            def _():
                pltpu.sync_copy(
                    acc.at[pl.ds(j * CH, CH)],
                    out_hbm.at[c, pl.ds(j * CH, CH)],
                )

    return k(h, src, dst, ew)


_BLK = 2000


def _tc_combine(p):
    def body(p_ref, o_ref):
        o_ref[...] = p_ref[0] + p_ref[1]

    return pl.pallas_call(
        body,
        out_shape=jax.ShapeDtypeStruct((N, D), jnp.float32),
        grid=(N // _BLK,),
        in_specs=[pl.BlockSpec((NC, _BLK, D), lambda i: (0, i, 0))],
        out_specs=pl.BlockSpec((_BLK, D), lambda i: (i, 0)),
    )(p)


def _tc_final(x, h1, q, W, b2):
    def body(x_ref, h1_ref, q_ref, w_ref, b_ref, o_ref):
        sm = (x_ref[...] + h1_ref[...] + q_ref[0] + q_ref[1]) * (1.0 / 3.0)
        o_ref[...] = (
            lax.dot_general(
                sm,
                w_ref[...],
                (((1,), (1,)), ((), ())),
                precision=lax.Precision.HIGHEST,
            )
            + b_ref[...]
        )

    return pl.pallas_call(
        body,
        out_shape=jax.ShapeDtypeStruct((N, D), jnp.float32),
        grid=(N // _BLK,),
        in_specs=[
            pl.BlockSpec((_BLK, D), lambda i: (i, 0)),
            pl.BlockSpec((_BLK, D), lambda i: (i, 0)),
            pl.BlockSpec((NC, _BLK, D), lambda i: (0, i, 0)),
            pl.BlockSpec((D, D), lambda i: (0, 0)),
            pl.BlockSpec((1, D), lambda i: (0, 0)),
        ],
        out_specs=pl.BlockSpec((_BLK, D), lambda i: (i, 0)),
    )(x, h1, q, W, b2)


def kernel(x, edge_index, edge_weight, W, b):
    src = edge_index[0].astype(jnp.int32)
    dst = edge_index[1].astype(jnp.int32)
    ew = edge_weight.astype(jnp.float32)
    p = _sc_round(x, src, dst, ew)
    h1 = _tc_combine(p)
    q = _sc_round(h1, src, dst, ew)
    return _tc_final(x, h1, q, W, b.reshape(1, D))
